# baseline (speedup 1.0000x reference)
import jax
import jax.numpy as jnp
from jax.experimental import pallas as pl
from jax.experimental.pallas import tpu as pltpu

K = 1024
D = 64
N = 16384
BN = 256


def _vq_kernel(zf_ref, zo_ref, emb16_ref, emb2x16_ref, embt_ref,
               enc_ref, zq_ref, idx_ref, loss_ref, perp_ref,
               counts_ref, sse_ref, e2_ref):
    step = pl.program_id(0)

    zf = zf_ref[...]

    w = zf * zf
    acc = w[:, 0:8]
    for i in range(1, 8):
        acc = acc + w[:, 8 * i:8 * i + 8]
    acc = acc[:, :4] + acc[:, 4:]
    acc = acc[:, :2] + acc[:, 2:]
    z2 = acc[:, 0:1] + acc[:, 1:2]

    @pl.when(step == 0)
    def _init():
        t = embt_ref[...] * embt_ref[...]
        te = t[0:8, :]
        for i in range(1, 8):
            te = te + t[8 * i:8 * i + 8, :]
        te = te[:4, :] + te[4:, :]
        te = te[:2, :] + te[2:, :]
        e2_ref[...] = te[0:1, :] + te[1:2, :]
        counts_ref[...] = jnp.zeros_like(counts_ref)
        sse_ref[...] = jnp.zeros_like(sse_ref)

    cross2 = jax.lax.dot_general(
        zf.astype(jnp.bfloat16), emb2x16_ref[...],
        (((1,), (1,)), ((), ())),
        preferred_element_type=jnp.float32)
    dist = z2 + e2_ref[...] - cross2

    lanes = jax.lax.broadcasted_iota(jnp.int32, (BN, K), 1)
    dmin = jnp.min(dist, axis=1, keepdims=True)
    idx = jnp.min(jnp.where(dist == dmin, lanes, K), axis=1).astype(jnp.int32)
    idx_ref[...] = idx[:, None]

    onehot = jnp.where(lanes == idx[:, None], 1.0, 0.0).astype(jnp.float32)
    enc_ref[...] = onehot

    oh16 = onehot.astype(jnp.bfloat16)
    zq = jax.lax.dot_general(
        oh16, emb16_ref[...],
        (((1,), (0,)), ((), ())),
        preferred_element_type=jnp.float32)
    zq_ref[...] = zq

    ones_row = jnp.ones((1, BN), jnp.bfloat16)
    counts_ref[...] += jax.lax.dot_general(
        ones_row, oh16, (((1,), (0,)), ((), ())),
        preferred_element_type=jnp.float32)
    diff = zq - zo_ref[...]
    sse_ref[...] += jnp.sum(diff * diff)[None, None]

    @pl.when(step == (N // BN) - 1)
    def _finish():
        loss_ref[...] = (1.5 * sse_ref[0, 0] / jnp.float32(N * D))[None, None]
        e_mean = counts_ref[...] / jnp.float32(N)
        ent = -jnp.sum(e_mean * jnp.log(e_mean + 1e-10))
        perp_ref[...] = jnp.exp(ent)[None, None]


def _vq_shard(z, emb_weight):
    B, C, H, W = z.shape
    n_loc = B * H * W
    nsteps = n_loc // BN
    z_flat = jnp.transpose(z, (0, 2, 3, 1)).reshape(n_loc, D)
    z_orig = z.reshape(n_loc, D)

    out = pl.pallas_call(
        _vq_kernel,
        grid=(nsteps,),
        in_specs=[
            pl.BlockSpec((BN, D), lambda i: (i, 0)),
            pl.BlockSpec((BN, D), lambda i: (i, 0)),
            pl.BlockSpec((K, D), lambda i: (0, 0)),
            pl.BlockSpec((K, D), lambda i: (0, 0)),
            pl.BlockSpec((D, K), lambda i: (0, 0)),
        ],
        out_specs=[
            pl.BlockSpec((BN, K), lambda i: (i, 0)),
            pl.BlockSpec((BN, D), lambda i: (i, 0)),
            pl.BlockSpec((BN, 1), lambda i: (i, 0)),
            pl.BlockSpec((1, 1), lambda i: (0, 0)),
            pl.BlockSpec((1, 1), lambda i: (0, 0)),
        ],
        out_shape=[
            jax.ShapeDtypeStruct((n_loc, K), jnp.float32),
            jax.ShapeDtypeStruct((n_loc, D), jnp.float32),
            jax.ShapeDtypeStruct((n_loc, 1), jnp.int32),
            jax.ShapeDtypeStruct((1, 1), jnp.float32),
            jax.ShapeDtypeStruct((1, 1), jnp.float32),
        ],
        scratch_shapes=[
            pltpu.VMEM((1, K), jnp.float32),
            pltpu.VMEM((1, 1), jnp.float32),
            pltpu.VMEM((1, K), jnp.float32),
        ],
    )(z_flat, z_orig,
      emb_weight.astype(jnp.bfloat16),
      (emb_weight * 2.0).astype(jnp.bfloat16),
      emb_weight.T)

    min_encodings, zq_flat, encoding_indices, loss, perplexity = out
    z_q = zq_flat.reshape(B, D, H, W)
    return (z_q, perplexity[0, 0], encoding_indices,
            min_encodings, loss[0, 0])


def kernel(z, emb_weight):
    return _vq_shard(z, emb_weight)


# speedup vs baseline: 1.0912x; 1.0912x over previous
import jax
import jax.numpy as jnp
from jax.experimental import pallas as pl
from jax.experimental.pallas import tpu as pltpu

K = 1024
D = 64
N = 16384
BN = 256


def _vq_kernel(zf_ref, zo_ref, emb_ref, embt_ref,
               enc_ref, zq_ref, idx_ref, loss_ref, perp_ref,
               counts_ref, sse_ref):
    step = pl.program_id(0)

    zf = zf_ref[...]
    emb = emb_ref[...]
    embt = embt_ref[...]

    w = zf * zf
    acc = w[:, 0:8]
    for i in range(1, 8):
        acc = acc + w[:, 8 * i:8 * i + 8]
    acc = acc[:, :4] + acc[:, 4:]
    acc = acc[:, :2] + acc[:, 2:]
    z2 = acc[:, 0:1] + acc[:, 1:2]

    t = embt * embt
    te = t[0:8, :]
    for i in range(1, 8):
        te = te + t[8 * i:8 * i + 8, :]
    te = te[:4, :] + te[4:, :]
    te = te[:2, :] + te[2:, :]
    e2 = te[0:1, :] + te[1:2, :]

    cross = jax.lax.dot_general(
        zf.astype(jnp.bfloat16), emb.astype(jnp.bfloat16),
        (((1,), (1,)), ((), ())),
        preferred_element_type=jnp.float32)
    dist = z2 + e2 - 2.0 * cross

    lanesf = jax.lax.broadcasted_iota(jnp.int32, (BN, K), 1).astype(jnp.float32)
    dmin = jnp.min(dist, axis=1, keepdims=True)
    idxf = jnp.min(jnp.where(dist == dmin, lanesf, jnp.float32(K)), axis=1)
    idx_ref[...] = idxf.astype(jnp.int32)[:, None]

    onehot = jnp.where(lanesf == idxf[:, None], 1.0, 0.0).astype(jnp.float32)
    enc_ref[...] = onehot

    oh16 = onehot.astype(jnp.bfloat16)
    zq = jax.lax.dot_general(
        oh16, emb.astype(jnp.bfloat16),
        (((1,), (0,)), ((), ())),
        preferred_element_type=jnp.float32)
    zq_ref[...] = zq

    @pl.when(step == 0)
    def _init():
        counts_ref[...] = jnp.zeros_like(counts_ref)
        sse_ref[...] = jnp.zeros_like(sse_ref)

    ones_row = jnp.ones((1, BN), jnp.bfloat16)
    counts_ref[...] += jax.lax.dot_general(
        ones_row, oh16, (((1,), (0,)), ((), ())),
        preferred_element_type=jnp.float32)
    diff = zq - zo_ref[...]
    sse_ref[...] += jnp.sum(diff * diff)[None, None]

    @pl.when(step == (N // BN) - 1)
    def _finish():
        loss_ref[...] = (1.5 * sse_ref[0, 0] / jnp.float32(N * D))[None, None]
        e_mean = counts_ref[...] / jnp.float32(N)
        ent = -jnp.sum(e_mean * jnp.log(e_mean + 1e-10))
        perp_ref[...] = jnp.exp(ent)[None, None]


def _vq_shard(z, emb_weight):
    B, C, H, W = z.shape
    n_loc = B * H * W
    nsteps = n_loc // BN
    z_flat = jnp.transpose(z, (0, 2, 3, 1)).reshape(n_loc, D)
    z_orig = z.reshape(n_loc, D)

    out = pl.pallas_call(
        _vq_kernel,
        grid=(nsteps,),
        in_specs=[
            pl.BlockSpec((BN, D), lambda i: (i, 0)),
            pl.BlockSpec((BN, D), lambda i: (i, 0)),
            pl.BlockSpec((K, D), lambda i: (0, 0)),
            pl.BlockSpec((D, K), lambda i: (0, 0)),
        ],
        out_specs=[
            pl.BlockSpec((BN, K), lambda i: (i, 0)),
            pl.BlockSpec((BN, D), lambda i: (i, 0)),
            pl.BlockSpec((BN, 1), lambda i: (i, 0)),
            pl.BlockSpec((1, 1), lambda i: (0, 0)),
            pl.BlockSpec((1, 1), lambda i: (0, 0)),
        ],
        out_shape=[
            jax.ShapeDtypeStruct((n_loc, K), jnp.float32),
            jax.ShapeDtypeStruct((n_loc, D), jnp.float32),
            jax.ShapeDtypeStruct((n_loc, 1), jnp.int32),
            jax.ShapeDtypeStruct((1, 1), jnp.float32),
            jax.ShapeDtypeStruct((1, 1), jnp.float32),
        ],
        scratch_shapes=[
            pltpu.VMEM((1, K), jnp.float32),
            pltpu.VMEM((1, 1), jnp.float32),
        ],
    )(z_flat, z_orig, emb_weight, emb_weight.T)

    min_encodings, zq_flat, encoding_indices, loss, perplexity = out
    z_q = zq_flat.reshape(B, D, H, W)
    return (z_q, perplexity[0, 0], encoding_indices,
            min_encodings, loss[0, 0])


def kernel(z, emb_weight):
    return _vq_shard(z, emb_weight)
